# baseline (speedup 1.0000x reference)
import functools

import jax
import jax.numpy as jnp
from jax import lax
from jax.experimental import pallas as pl
from jax.experimental.pallas import tpu as pltpu
from jax.experimental.pallas import tpu_sc as plsc

B = 128
ENC_LEN = 50
MAX_LOCAL = 512
EMBED_UNITS = 300
TRANS_UNITS = 100
WORD_VOCAB = 30000
ENT_VOCAB = 100007

NC = 2
NS = 16
NW = NC * NS

W_ROWS = B * ENC_LEN
E_ROWS = B * MAX_LOCAL
E_CHUNK = 4096
E_NCHUNK = E_ROWS // E_CHUNK


def _gather16(row_v, idx_v, out_v, n):
    def body(i, _):
        off = pl.multiple_of(i * 256, 256)
        ivs = [idx_v[pl.ds(off + 16 * j, 16)] for j in range(16)]
        gs = [plsc.load_gather(row_v, [iv]) for iv in ivs]
        for j in range(16):
            out_v[pl.ds(off + 16 * j, 16)] = gs[j]
        return 0

    lax.fori_loop(0, n // 256, body, 0)


def _gather_word_body(widx_hbm, wtab_hbm, wout_hbm,
                      idx_v, row0_v, row1_v, out0_v, out1_v, sem_r, sem_o):
    wid = lax.axis_index("s") * NC + lax.axis_index("c")
    pltpu.sync_copy(widx_hbm, idx_v)
    f0 = jnp.minimum(wid, EMBED_UNITS - 1)
    row_cp = [None, None]
    out_cp = [None, None]
    rows_v = (row0_v, row1_v)
    outs_v = (out0_v, out1_v)
    row_cp[0] = pltpu.async_copy(wtab_hbm.at[f0], rows_v[0], sem_r[0])
    for k in range(10):
        cur, nxt = k % 2, (k + 1) % 2
        f = jnp.minimum(wid + NW * k, EMBED_UNITS - 1)
        if k + 1 < 10:
            fn = jnp.minimum(wid + NW * (k + 1), EMBED_UNITS - 1)
            row_cp[nxt] = pltpu.async_copy(wtab_hbm.at[fn], rows_v[nxt],
                                           sem_r[nxt])
        row_cp[cur].wait()
        if out_cp[cur] is not None:
            out_cp[cur].wait()
        _gather16(rows_v[cur], idx_v, outs_v[cur], W_ROWS)
        out_cp[cur] = pltpu.async_copy(outs_v[cur], wout_hbm.at[f],
                                       sem_o[cur])
    for cp in out_cp:
        cp.wait()


_gather_word = functools.partial(
    pl.kernel,
    out_type=jax.ShapeDtypeStruct((EMBED_UNITS, W_ROWS), jnp.float32),
    mesh=plsc.VectorSubcoreMesh(core_axis_name="c", subcore_axis_name="s"),
    compiler_params=pltpu.CompilerParams(needs_layout_passes=False),
    scratch_types=[
        pltpu.VMEM((W_ROWS,), jnp.int32),
        pltpu.VMEM((WORD_VOCAB,), jnp.float32),
        pltpu.VMEM((WORD_VOCAB,), jnp.float32),
        pltpu.VMEM((W_ROWS,), jnp.float32),
        pltpu.VMEM((W_ROWS,), jnp.float32),
        (pltpu.SemaphoreType.DMA, pltpu.SemaphoreType.DMA),
        (pltpu.SemaphoreType.DMA, pltpu.SemaphoreType.DMA),
    ],
)(_gather_word_body)


def _gather_ent_body(eidx_hbm, etab_hbm, eout_hbm,
                     idx0_v, idx1_v, row_v, out0_v, out1_v, sem_i, sem_o):
    wid = lax.axis_index("s") * NC + lax.axis_index("c")
    for k in range(4):
        f = wid + NW * k

        @pl.when(f < TRANS_UNITS)
        def _():
            pltpu.sync_copy(etab_hbm.at[f], row_v)
            idx_cp = [None, None]
            out_cp = [None, None]
            idxs_v = (idx0_v, idx1_v)
            outs_v = (out0_v, out1_v)
            idx_cp[0] = pltpu.async_copy(eidx_hbm.at[pl.ds(0, E_CHUNK)],
                                         idxs_v[0], sem_i[0])
            for c in range(E_NCHUNK):
                cur, nxt = c % 2, (c + 1) % 2
                if c + 1 < E_NCHUNK:
                    idx_cp[nxt] = pltpu.async_copy(
                        eidx_hbm.at[pl.ds((c + 1) * E_CHUNK, E_CHUNK)],
                        idxs_v[nxt], sem_i[nxt])
                idx_cp[cur].wait()
                if out_cp[cur] is not None:
                    out_cp[cur].wait()
                _gather16(row_v, idxs_v[cur], outs_v[cur], E_CHUNK)
                out_cp[cur] = pltpu.async_copy(
                    outs_v[cur],
                    eout_hbm.at[f, pl.ds(c * E_CHUNK, E_CHUNK)], sem_o[cur])
            for cp in out_cp:
                cp.wait()


_gather_ent = functools.partial(
    pl.kernel,
    out_type=jax.ShapeDtypeStruct((TRANS_UNITS, E_ROWS), jnp.float32),
    mesh=plsc.VectorSubcoreMesh(core_axis_name="c", subcore_axis_name="s"),
    compiler_params=pltpu.CompilerParams(needs_layout_passes=False),
    scratch_types=[
        pltpu.VMEM((E_CHUNK,), jnp.int32),
        pltpu.VMEM((E_CHUNK,), jnp.int32),
        pltpu.VMEM((ENT_VOCAB,), jnp.float32),
        pltpu.VMEM((E_CHUNK,), jnp.float32),
        pltpu.VMEM((E_CHUNK,), jnp.float32),
        (pltpu.SemaphoreType.DMA, pltpu.SemaphoreType.DMA),
        (pltpu.SemaphoreType.DMA, pltpu.SemaphoreType.DMA),
    ],
)(_gather_ent_body)


def _lstm_body(xw_ref, wih_ref, whh_ref, b_ref, out_ref, xp_ref):
    xp_ref[...] = jax.lax.dot_general(
        wih_ref[...], xw_ref[...], (((1,), (0,)), ((), ())),
        preferred_element_type=jnp.float32)
    h = jnp.zeros((128, B), jnp.float32)
    c = jnp.zeros((128, B), jnp.float32)
    for t in range(ENC_LEN):
        g = (xp_ref[:, t * B:(t + 1) * B]
             + jax.lax.dot_general(whh_ref[...], h, (((1,), (0,)), ((), ())),
                                   preferred_element_type=jnp.float32)
             + b_ref[...])
        i = jax.nn.sigmoid(g[0:128, :])
        f = jax.nn.sigmoid(g[128:256, :])
        gg = jnp.tanh(g[256:384, :])
        o = jax.nn.sigmoid(g[384:512, :])
        c = f * c + i * gg
        h = o * jnp.tanh(c)
        out_ref[t] = h[0:TRANS_UNITS, :]


def _entlin_body(rows_ref, w_ref, b_ref, out_ref):
    res = jnp.maximum(
        jax.lax.dot_general(w_ref[...], rows_ref[...], (((1,), (0,)), ((), ())),
                            preferred_element_type=jnp.float32)
        + b_ref[...], 0.0)
    out_ref[...] = res[:, None, None, :]


def kernel(query_text, answer_text, local_entity, q2e_adj_mat, kb_fact_rel,
           match_entity_one_hop, only_two_entity, match_entity_only_two,
           one_two_triples_id, posts_length, responses_length,
           word_embed, entity_embed, lstm_Wih, lstm_Whh, lstm_b,
           entity_W, entity_b):
    local_entity_mask = (local_entity != 0).astype(jnp.float32)
    query_mask = (query_text != 0).astype(jnp.float32)
    pagerank_f = q2e_adj_mat
    responses_id = jnp.concatenate(
        [jnp.ones((B, 1), answer_text.dtype), answer_text[:, :-1]], axis=1)

    widx = jnp.transpose(query_text).reshape(-1)
    eidx = local_entity.reshape(-1)
    xwT = _gather_word(widx, jnp.transpose(word_embed))
    gT = _gather_ent(eidx, jnp.transpose(entity_embed))

    wih_p = jnp.pad(lstm_Wih.reshape(4, TRANS_UNITS, EMBED_UNITS),
                    ((0, 0), (0, 28), (0, 0))).reshape(512, EMBED_UNITS)
    whh_p = jnp.pad(lstm_Whh.reshape(4, TRANS_UNITS, TRANS_UNITS),
                    ((0, 0), (0, 28), (0, 28))).reshape(4 * 128, 128)
    b_p = jnp.pad(lstm_b.reshape(4, TRANS_UNITS),
                  ((0, 0), (0, 28))).reshape(512, 1)

    hsT = pl.pallas_call(
        _lstm_body,
        out_shape=jax.ShapeDtypeStruct((ENC_LEN, TRANS_UNITS, B), jnp.float32),
        scratch_shapes=[pltpu.VMEM((512, W_ROWS), jnp.float32)],
    )(xwT, wih_p, whh_p, b_p)
    query_hidden_emb = jnp.transpose(hsT, (2, 0, 1))
    query_node_emb = jnp.transpose(hsT[-1], (1, 0))[None]

    entT = pl.pallas_call(
        _entlin_body,
        grid=(B,),
        in_specs=[
            pl.BlockSpec((TRANS_UNITS, MAX_LOCAL), lambda i: (0, i)),
            pl.BlockSpec((TRANS_UNITS, TRANS_UNITS), lambda i: (0, 0)),
            pl.BlockSpec((TRANS_UNITS, 1), lambda i: (0, 0)),
        ],
        out_specs=pl.BlockSpec((TRANS_UNITS, 1, 1, MAX_LOCAL),
                               lambda i: (0, i, 0, 0)),
        out_shape=jax.ShapeDtypeStruct((TRANS_UNITS, B, 1, MAX_LOCAL),
                                       jnp.float32),
    )(gT, entity_W, entity_b[:, None])
    local_entity_emb = jnp.transpose(entT[:, :, 0, :], (1, 2, 0))

    return (query_hidden_emb, query_node_emb, local_entity_emb,
            local_entity_mask, query_mask, responses_id, pagerank_f)


# speedup vs baseline: 1.5034x; 1.5034x over previous
import functools

import jax
import jax.numpy as jnp
from jax import lax
from jax.experimental import pallas as pl
from jax.experimental.pallas import tpu as pltpu
from jax.experimental.pallas import tpu_sc as plsc

B = 128
ENC_LEN = 50
MAX_LOCAL = 512
EMBED_UNITS = 300
TRANS_UNITS = 100
WORD_VOCAB = 30000
ENT_VOCAB = 100007

NC = 2
NS = 16
NW = NC * NS

W_ROWS = B * ENC_LEN
E_ROWS = B * MAX_LOCAL
E_CHUNK = 4096
E_NCHUNK = E_ROWS // E_CHUNK


def _gather16(row_v, idx_v, out_v, n):
    def body(i, _):
        off = pl.multiple_of(i * 256, 256)
        ivs = [idx_v[pl.ds(off + 16 * j, 16)] for j in range(16)]
        gs = [plsc.load_gather(row_v, [iv]) for iv in ivs]
        for j in range(16):
            out_v[pl.ds(off + 16 * j, 16)] = gs[j]
        return 0

    lax.fori_loop(0, n // 256, body, 0)


def _gather_word_body(widx_hbm, wtab_hbm, wout_hbm,
                      idx_v, row0_v, row1_v, out0_v, out1_v, sem_r, sem_o):
    wid = lax.axis_index("s") * NC + lax.axis_index("c")
    pltpu.sync_copy(widx_hbm, idx_v)
    f0 = jnp.minimum(wid, EMBED_UNITS - 1)
    row_cp = [None, None]
    out_cp = [None, None]
    rows_v = (row0_v, row1_v)
    outs_v = (out0_v, out1_v)
    row_cp[0] = pltpu.async_copy(wtab_hbm.at[f0], rows_v[0], sem_r[0])
    for k in range(10):
        cur, nxt = k % 2, (k + 1) % 2
        f = jnp.minimum(wid + NW * k, EMBED_UNITS - 1)
        if k + 1 < 10:
            fn = jnp.minimum(wid + NW * (k + 1), EMBED_UNITS - 1)
            row_cp[nxt] = pltpu.async_copy(wtab_hbm.at[fn], rows_v[nxt],
                                           sem_r[nxt])
        row_cp[cur].wait()
        if out_cp[cur] is not None:
            out_cp[cur].wait()
        _gather16(rows_v[cur], idx_v, outs_v[cur], W_ROWS)
        out_cp[cur] = pltpu.async_copy(outs_v[cur], wout_hbm.at[f],
                                       sem_o[cur])
    for cp in out_cp:
        cp.wait()


_gather_word = functools.partial(
    pl.kernel,
    out_type=jax.ShapeDtypeStruct((EMBED_UNITS, W_ROWS), jnp.float32),
    mesh=plsc.VectorSubcoreMesh(core_axis_name="c", subcore_axis_name="s"),
    compiler_params=pltpu.CompilerParams(needs_layout_passes=False),
    scratch_types=[
        pltpu.VMEM((W_ROWS,), jnp.int32),
        pltpu.VMEM((WORD_VOCAB,), jnp.float32),
        pltpu.VMEM((WORD_VOCAB,), jnp.float32),
        pltpu.VMEM((W_ROWS,), jnp.float32),
        pltpu.VMEM((W_ROWS,), jnp.float32),
        (pltpu.SemaphoreType.DMA, pltpu.SemaphoreType.DMA),
        (pltpu.SemaphoreType.DMA, pltpu.SemaphoreType.DMA),
    ],
)(_gather_word_body)


def _gather_ent_body(eidx_hbm, etab_hbm, eout_hbm,
                     idx0_v, idx1_v, row_v, out0_v, out1_v, sem_i, sem_o):
    wid = lax.axis_index("s") * NC + lax.axis_index("c")
    for k in range(4):
        f = wid + NW * k

        @pl.when(f < TRANS_UNITS)
        def _():
            pltpu.sync_copy(etab_hbm.at[f], row_v)
            idx_cp = [None, None]
            out_cp = [None, None]
            idxs_v = (idx0_v, idx1_v)
            outs_v = (out0_v, out1_v)
            idx_cp[0] = pltpu.async_copy(eidx_hbm.at[pl.ds(0, E_CHUNK)],
                                         idxs_v[0], sem_i[0])
            for c in range(E_NCHUNK):
                cur, nxt = c % 2, (c + 1) % 2
                if c + 1 < E_NCHUNK:
                    idx_cp[nxt] = pltpu.async_copy(
                        eidx_hbm.at[pl.ds((c + 1) * E_CHUNK, E_CHUNK)],
                        idxs_v[nxt], sem_i[nxt])
                idx_cp[cur].wait()
                if out_cp[cur] is not None:
                    out_cp[cur].wait()
                _gather16(row_v, idxs_v[cur], outs_v[cur], E_CHUNK)
                out_cp[cur] = pltpu.async_copy(
                    outs_v[cur],
                    eout_hbm.at[f, pl.ds(c * E_CHUNK, E_CHUNK)], sem_o[cur])
            for cp in out_cp:
                cp.wait()


_gather_ent = functools.partial(
    pl.kernel,
    out_type=jax.ShapeDtypeStruct((TRANS_UNITS, E_ROWS), jnp.float32),
    mesh=plsc.VectorSubcoreMesh(core_axis_name="c", subcore_axis_name="s"),
    compiler_params=pltpu.CompilerParams(needs_layout_passes=False),
    scratch_types=[
        pltpu.VMEM((E_CHUNK,), jnp.int32),
        pltpu.VMEM((E_CHUNK,), jnp.int32),
        pltpu.VMEM((ENT_VOCAB,), jnp.float32),
        pltpu.VMEM((E_CHUNK,), jnp.float32),
        pltpu.VMEM((E_CHUNK,), jnp.float32),
        (pltpu.SemaphoreType.DMA, pltpu.SemaphoreType.DMA),
        (pltpu.SemaphoreType.DMA, pltpu.SemaphoreType.DMA),
    ],
)(_gather_ent_body)


def _lstm_body(xw_ref, wih_ref, whh_ref, b_ref, out_ref, xp_ref):
    xp_ref[...] = jax.lax.dot_general(
        wih_ref[...], xw_ref[...], (((1,), (0,)), ((), ())),
        preferred_element_type=jnp.float32)
    h = jnp.zeros((128, B), jnp.float32)
    c = jnp.zeros((128, B), jnp.float32)
    for t in range(ENC_LEN):
        g = (xp_ref[:, t * B:(t + 1) * B]
             + jax.lax.dot_general(whh_ref[...], h, (((1,), (0,)), ((), ())),
                                   preferred_element_type=jnp.float32)
             + b_ref[...])
        i = jax.nn.sigmoid(g[0:128, :])
        f = jax.nn.sigmoid(g[128:256, :])
        gg = jnp.tanh(g[256:384, :])
        o = jax.nn.sigmoid(g[384:512, :])
        c = f * c + i * gg
        h = o * jnp.tanh(c)
        out_ref[t] = h[0:TRANS_UNITS, :]


def _entlin_body(rows_ref, w_ref, b_ref, out_ref):
    res = jnp.maximum(
        jax.lax.dot_general(w_ref[...], rows_ref[...], (((1,), (0,)), ((), ())),
                            preferred_element_type=jnp.float32)
        + b_ref[...], 0.0)
    for j in range(16):
        out_ref[:, j, :] = res[:, j * MAX_LOCAL:(j + 1) * MAX_LOCAL]


def kernel(query_text, answer_text, local_entity, q2e_adj_mat, kb_fact_rel,
           match_entity_one_hop, only_two_entity, match_entity_only_two,
           one_two_triples_id, posts_length, responses_length,
           word_embed, entity_embed, lstm_Wih, lstm_Whh, lstm_b,
           entity_W, entity_b):
    local_entity_mask = (local_entity != 0).astype(jnp.float32)
    query_mask = (query_text != 0).astype(jnp.float32)
    pagerank_f = q2e_adj_mat
    responses_id = jnp.concatenate(
        [jnp.ones((B, 1), answer_text.dtype), answer_text[:, :-1]], axis=1)

    widx = jnp.transpose(query_text).reshape(-1)
    eidx = local_entity.reshape(-1)
    xwT = _gather_word(widx, jnp.transpose(word_embed))
    gT = _gather_ent(eidx, jnp.transpose(entity_embed))

    wih_p = jnp.pad(lstm_Wih.reshape(4, TRANS_UNITS, EMBED_UNITS),
                    ((0, 0), (0, 28), (0, 0))).reshape(512, EMBED_UNITS)
    whh_p = jnp.pad(lstm_Whh.reshape(4, TRANS_UNITS, TRANS_UNITS),
                    ((0, 0), (0, 28), (0, 28))).reshape(4 * 128, 128)
    b_p = jnp.pad(lstm_b.reshape(4, TRANS_UNITS),
                  ((0, 0), (0, 28))).reshape(512, 1)

    hsT = pl.pallas_call(
        _lstm_body,
        out_shape=jax.ShapeDtypeStruct((ENC_LEN, TRANS_UNITS, B), jnp.float32),
        scratch_shapes=[pltpu.VMEM((512, W_ROWS), jnp.float32)],
    )(xwT, wih_p, whh_p, b_p)
    query_hidden_emb = jnp.transpose(hsT, (2, 0, 1))
    query_node_emb = jnp.transpose(hsT[-1], (1, 0))[None]

    entT = pl.pallas_call(
        _entlin_body,
        grid=(B // 16,),
        in_specs=[
            pl.BlockSpec((TRANS_UNITS, 16 * MAX_LOCAL), lambda i: (0, i)),
            pl.BlockSpec((TRANS_UNITS, TRANS_UNITS), lambda i: (0, 0)),
            pl.BlockSpec((TRANS_UNITS, 1), lambda i: (0, 0)),
        ],
        out_specs=pl.BlockSpec((TRANS_UNITS, 16, MAX_LOCAL),
                               lambda i: (0, i, 0)),
        out_shape=jax.ShapeDtypeStruct((TRANS_UNITS, B, MAX_LOCAL),
                                       jnp.float32),
    )(gT, entity_W, entity_b[:, None])
    local_entity_emb = jnp.transpose(entT, (1, 2, 0))

    return (query_hidden_emb, query_node_emb, local_entity_emb,
            local_entity_mask, query_mask, responses_id, pagerank_f)


# speedup vs baseline: 1.5094x; 1.0040x over previous
import functools

import jax
import jax.numpy as jnp
from jax import lax
from jax.experimental import pallas as pl
from jax.experimental.pallas import tpu as pltpu
from jax.experimental.pallas import tpu_sc as plsc

B = 128
ENC_LEN = 50
MAX_LOCAL = 512
EMBED_UNITS = 300
TRANS_UNITS = 100
WORD_VOCAB = 30000
ENT_VOCAB = 100007

NC = 2
NS = 16
NW = NC * NS

W_ROWS = B * ENC_LEN
E_ROWS = B * MAX_LOCAL
E_CHUNK = 4096
E_NCHUNK = E_ROWS // E_CHUNK


def _gather16(row_v, idx_v, out_v, n):
    def body(i, _):
        off = pl.multiple_of(i * 256, 256)
        ivs = [idx_v[pl.ds(off + 16 * j, 16)] for j in range(16)]
        gs = [plsc.load_gather(row_v, [iv]) for iv in ivs]
        for j in range(16):
            out_v[pl.ds(off + 16 * j, 16)] = gs[j]
        return 0

    lax.fori_loop(0, n // 256, body, 0)


def _gather_word_body(widx_hbm, wtab_hbm, wout_hbm,
                      idx_v, row0_v, row1_v, out0_v, out1_v, sem_r, sem_o):
    wid = lax.axis_index("s") * NC + lax.axis_index("c")
    pltpu.sync_copy(widx_hbm, idx_v)
    f0 = jnp.minimum(wid, EMBED_UNITS - 1)
    row_cp = [None, None]
    out_cp = [None, None]
    rows_v = (row0_v, row1_v)
    outs_v = (out0_v, out1_v)
    row_cp[0] = pltpu.async_copy(wtab_hbm.at[f0], rows_v[0], sem_r[0])
    for k in range(10):
        cur, nxt = k % 2, (k + 1) % 2
        f = jnp.minimum(wid + NW * k, EMBED_UNITS - 1)
        if k + 1 < 10:
            fn = jnp.minimum(wid + NW * (k + 1), EMBED_UNITS - 1)
            row_cp[nxt] = pltpu.async_copy(wtab_hbm.at[fn], rows_v[nxt],
                                           sem_r[nxt])
        row_cp[cur].wait()
        if out_cp[cur] is not None:
            out_cp[cur].wait()
        _gather16(rows_v[cur], idx_v, outs_v[cur], W_ROWS)
        out_cp[cur] = pltpu.async_copy(outs_v[cur], wout_hbm.at[f],
                                       sem_o[cur])
    for cp in out_cp:
        cp.wait()


_gather_word = functools.partial(
    pl.kernel,
    out_type=jax.ShapeDtypeStruct((EMBED_UNITS, W_ROWS), jnp.float32),
    mesh=plsc.VectorSubcoreMesh(core_axis_name="c", subcore_axis_name="s"),
    compiler_params=pltpu.CompilerParams(needs_layout_passes=False),
    scratch_types=[
        pltpu.VMEM((W_ROWS,), jnp.int32),
        pltpu.VMEM((WORD_VOCAB,), jnp.float32),
        pltpu.VMEM((WORD_VOCAB,), jnp.float32),
        pltpu.VMEM((W_ROWS,), jnp.float32),
        pltpu.VMEM((W_ROWS,), jnp.float32),
        (pltpu.SemaphoreType.DMA, pltpu.SemaphoreType.DMA),
        (pltpu.SemaphoreType.DMA, pltpu.SemaphoreType.DMA),
    ],
)(_gather_word_body)


def _gather_ent_body(eidx_hbm, etab_hbm, eout_hbm,
                     idx0_v, idx1_v, row_v, out0_v, out1_v, sem_i, sem_o):
    wid = lax.axis_index("s") * NC + lax.axis_index("c")
    for k in range(4):
        f = wid + NW * k

        @pl.when(f < TRANS_UNITS)
        def _():
            pltpu.sync_copy(etab_hbm.at[f], row_v)
            idx_cp = [None, None]
            out_cp = [None, None]
            idxs_v = (idx0_v, idx1_v)
            outs_v = (out0_v, out1_v)
            idx_cp[0] = pltpu.async_copy(eidx_hbm.at[pl.ds(0, E_CHUNK)],
                                         idxs_v[0], sem_i[0])
            for c in range(E_NCHUNK):
                cur, nxt = c % 2, (c + 1) % 2
                if c + 1 < E_NCHUNK:
                    idx_cp[nxt] = pltpu.async_copy(
                        eidx_hbm.at[pl.ds((c + 1) * E_CHUNK, E_CHUNK)],
                        idxs_v[nxt], sem_i[nxt])
                idx_cp[cur].wait()
                if out_cp[cur] is not None:
                    out_cp[cur].wait()
                _gather16(row_v, idxs_v[cur], outs_v[cur], E_CHUNK)
                out_cp[cur] = pltpu.async_copy(
                    outs_v[cur],
                    eout_hbm.at[f, pl.ds(c * E_CHUNK, E_CHUNK)], sem_o[cur])
            for cp in out_cp:
                cp.wait()


_gather_ent = functools.partial(
    pl.kernel,
    out_type=jax.ShapeDtypeStruct((TRANS_UNITS, E_ROWS), jnp.float32),
    mesh=plsc.VectorSubcoreMesh(core_axis_name="c", subcore_axis_name="s"),
    compiler_params=pltpu.CompilerParams(needs_layout_passes=False),
    scratch_types=[
        pltpu.VMEM((E_CHUNK,), jnp.int32),
        pltpu.VMEM((E_CHUNK,), jnp.int32),
        pltpu.VMEM((ENT_VOCAB,), jnp.float32),
        pltpu.VMEM((E_CHUNK,), jnp.float32),
        pltpu.VMEM((E_CHUNK,), jnp.float32),
        (pltpu.SemaphoreType.DMA, pltpu.SemaphoreType.DMA),
        (pltpu.SemaphoreType.DMA, pltpu.SemaphoreType.DMA),
    ],
)(_gather_ent_body)


def _lstm_body(xw_ref, wih_ref, whh_ref, b_ref, out_ref, xp_ref):
    xp_ref[...] = jax.lax.dot_general(
        wih_ref[...], xw_ref[...], (((1,), (0,)), ((), ())),
        preferred_element_type=jnp.float32)
    h = jnp.zeros((128, B), jnp.float32)
    c = jnp.zeros((128, B), jnp.float32)
    for t in range(ENC_LEN):
        g = (xp_ref[:, t * B:(t + 1) * B]
             + jax.lax.dot_general(whh_ref[...], h, (((1,), (0,)), ((), ())),
                                   preferred_element_type=jnp.float32)
             + b_ref[...])
        i = jax.nn.sigmoid(g[0:128, :])
        f = jax.nn.sigmoid(g[128:256, :])
        gg = jnp.tanh(g[256:384, :])
        o = jax.nn.sigmoid(g[384:512, :])
        c = f * c + i * gg
        h = o * jnp.tanh(c)
        out_ref[t] = h[0:TRANS_UNITS, :]


def _entlin_body(rows_ref, w_ref, b_ref, out_ref):
    res = jnp.maximum(
        jax.lax.dot_general(w_ref[...], rows_ref[...], (((1,), (0,)), ((), ())),
                            preferred_element_type=jnp.float32)
        + b_ref[...], 0.0)
    for j in range(16):
        out_ref[:, j, :] = res[:, j * MAX_LOCAL:(j + 1) * MAX_LOCAL]


def kernel(query_text, answer_text, local_entity, q2e_adj_mat, kb_fact_rel,
           match_entity_one_hop, only_two_entity, match_entity_only_two,
           one_two_triples_id, posts_length, responses_length,
           word_embed, entity_embed, lstm_Wih, lstm_Whh, lstm_b,
           entity_W, entity_b):
    local_entity_mask = (local_entity != 0).astype(jnp.float32)
    query_mask = (query_text != 0).astype(jnp.float32)
    pagerank_f = q2e_adj_mat
    responses_id = jnp.concatenate(
        [jnp.ones((B, 1), answer_text.dtype), answer_text[:, :-1]], axis=1)

    widx = jnp.transpose(query_text).reshape(-1)
    eidx = local_entity.reshape(-1)
    gT = _gather_ent(eidx, jnp.transpose(entity_embed))
    xwT = _gather_word(widx, jnp.transpose(word_embed))

    wih_p = jnp.pad(lstm_Wih.reshape(4, TRANS_UNITS, EMBED_UNITS),
                    ((0, 0), (0, 28), (0, 0))).reshape(512, EMBED_UNITS)
    whh_p = jnp.pad(lstm_Whh.reshape(4, TRANS_UNITS, TRANS_UNITS),
                    ((0, 0), (0, 28), (0, 28))).reshape(4 * 128, 128)
    b_p = jnp.pad(lstm_b.reshape(4, TRANS_UNITS),
                  ((0, 0), (0, 28))).reshape(512, 1)

    hsT = pl.pallas_call(
        _lstm_body,
        out_shape=jax.ShapeDtypeStruct((ENC_LEN, TRANS_UNITS, B), jnp.float32),
        scratch_shapes=[pltpu.VMEM((512, W_ROWS), jnp.float32)],
    )(xwT, wih_p, whh_p, b_p)
    query_hidden_emb = jnp.transpose(hsT, (2, 0, 1))
    query_node_emb = jnp.transpose(hsT[-1], (1, 0))[None]

    entT = pl.pallas_call(
        _entlin_body,
        grid=(B // 16,),
        in_specs=[
            pl.BlockSpec((TRANS_UNITS, 16 * MAX_LOCAL), lambda i: (0, i)),
            pl.BlockSpec((TRANS_UNITS, TRANS_UNITS), lambda i: (0, 0)),
            pl.BlockSpec((TRANS_UNITS, 1), lambda i: (0, 0)),
        ],
        out_specs=pl.BlockSpec((TRANS_UNITS, 16, MAX_LOCAL),
                               lambda i: (0, i, 0)),
        out_shape=jax.ShapeDtypeStruct((TRANS_UNITS, B, MAX_LOCAL),
                                       jnp.float32),
    )(gT, entity_W, entity_b[:, None])
    local_entity_emb = jnp.transpose(entT, (1, 2, 0))

    return (query_hidden_emb, query_node_emb, local_entity_emb,
            local_entity_mask, query_mask, responses_id, pagerank_f)
